# baseline (speedup 1.0000x reference)
import functools

import jax
import jax.numpy as jnp
from jax import lax
from jax.experimental import pallas as pl
from jax.experimental.pallas import tpu as pltpu
from jax.experimental.pallas import tpu_sc as plsc

K_HOPS = 10


def _mlp_body(ft_ref, w1_ref, b1_ref, w2_ref, b2_ref, o_ref):
    h = jnp.dot(w1_ref[...], ft_ref[...], preferred_element_type=jnp.float32)
    h = jnp.maximum(h + b1_ref[...], 0.0)
    o_ref[...] = (
        jnp.dot(w2_ref[...], h, preferred_element_type=jnp.float32) + b2_ref[...]
    )


def _comb_body(xs_ref, wa_ref, ba_ref, o_ref):
    xs = xs_ref[...]
    wa = wa_ref[...]
    s = jnp.sum(xs * wa[0][None, :, None], axis=1) + ba_ref[0, 0]
    score = jax.nn.sigmoid(s)
    out = jnp.sum(score[:, None, :] * xs, axis=0)
    m = jnp.max(out, axis=0, keepdims=True)
    lse = jnp.log(jnp.sum(jnp.exp(out - m), axis=0, keepdims=True))
    o_ref[...] = out - m - lse


def _make_sc_body(n_nodes, n_feat_tile, n_chunks, chunk, num_cores, n_tiles):
    slab = n_feat_tile * n_nodes
    n_pairs = n_chunks // 2
    shift = (n_nodes - 1).bit_length()

    def sc_body(x0_hbm, pk_hbm, w_hbm, xs_hbm, xa, xb, pkb, wb, sem0, sem1):
        wid = lax.axis_index("s") * num_cores + lax.axis_index("c")
        base = wid * slab
        sems = (sem0, sem1)

        def start(c, b):
            off = c * chunk
            boff = b * chunk
            pltpu.async_copy(
                pk_hbm.at[pl.ds(off, chunk)], pkb.at[pl.ds(boff, chunk)], sems[b]
            )
            pltpu.async_copy(
                w_hbm.at[pl.ds(off, chunk)], wb.at[pl.ds(boff, chunk)], sems[b]
            )

        def wait(b):
            boff = b * chunk
            for hbm, buf in ((pk_hbm, pkb), (w_hbm, wb)):
                pltpu.make_async_copy(
                    hbm.at[pl.ds(0, chunk)], buf.at[pl.ds(boff, chunk)], sems[b]
                ).wait()

        pltpu.sync_copy(x0_hbm.at[pl.ds(base, slab)], xa)
        pltpu.sync_copy(xa, xs_hbm.at[pl.ds(base, slab)])

        start(0, 0)
        start(1, 1)

        for k in range(1, K_HOPS + 1):
            src, dst = (xa, xb) if k % 2 == 1 else (xb, xa)

            srcf = [src.at[pl.ds(f * n_nodes, n_nodes)] for f in range(n_feat_tile)]
            dstf = [dst.at[pl.ds(f * n_nodes, n_nodes)] for f in range(n_feat_tile)]

            @plsc.parallel_loop(0, slab // 16, unroll=8)
            def _zero(i, dst=dst):
                dst[pl.ds(i * 16, 16)] = jnp.zeros((16,), jnp.float32)

            @pl.loop(0, n_pairs)
            def _pair(p, srcf=srcf, dstf=dstf):
                for b in range(2):
                    wait(b)

                    @plsc.parallel_loop(0, chunk // 16, unroll=16)
                    def _grp(g, srcf=srcf, dstf=dstf, b=b):
                        gb = g * 16 + b * chunk
                        pk16 = pkb[pl.ds(gb, 16)]
                        w16 = wb[pl.ds(gb, 16)]
                        col16 = lax.shift_right_logical(pk16, shift)
                        row16 = lax.bitwise_and(pk16, (1 << shift) - 1)
                        for f in range(n_feat_tile):
                            v = plsc.load_gather(srcf[f], [col16])
                            plsc.addupdate_scatter(dstf[f], [row16], v * w16)

                    start(lax.rem(2 * p + 2 + b, n_chunks), b)

            pltpu.sync_copy(dst, xs_hbm.at[pl.ds(k * (n_tiles * slab) + base, slab)])

        wait(0)
        wait(1)

    return sc_body


def kernel(feature, edge_index, edge_weight, W1, b1, W2, b2, Wa, ba):
    n_nodes, _ = feature.shape
    n_class = W2.shape[0]
    n_edges = edge_weight.shape[0]

    f_t = feature.T
    x0_t = pl.pallas_call(
        _mlp_body,
        out_shape=jax.ShapeDtypeStruct((n_class, n_nodes), jnp.float32),
    )(f_t, W1, b1[:, None], W2, b2[:, None])

    num_cores, num_subcores = 2, 16
    n_tiles = num_cores * num_subcores
    n_feat_tile = n_class // n_tiles

    chunk = 8000
    n_chunks = n_edges // chunk
    slab = n_feat_tile * n_nodes

    row = edge_index[0].astype(jnp.int32)
    col = edge_index[1].astype(jnp.int32)
    shift = (n_nodes - 1).bit_length()
    packed = col * (1 << shift) + row
    packed, w = lax.sort((packed, edge_weight.astype(jnp.float32)), num_keys=1)

    mesh = plsc.VectorSubcoreMesh(core_axis_name="c", subcore_axis_name="s")
    xs_flat = pl.kernel(
        _make_sc_body(n_nodes, n_feat_tile, n_chunks, chunk, num_cores, n_tiles),
        out_type=jax.ShapeDtypeStruct(((K_HOPS + 1) * n_class * n_nodes,), jnp.float32),
        mesh=mesh,
        compiler_params=pltpu.CompilerParams(needs_layout_passes=False),
        scratch_types=[
            pltpu.VMEM((slab,), jnp.float32),
            pltpu.VMEM((slab,), jnp.float32),
            pltpu.VMEM((2 * chunk,), jnp.int32),
            pltpu.VMEM((2 * chunk,), jnp.float32),
            pltpu.SemaphoreType.DMA,
            pltpu.SemaphoreType.DMA,
        ],
    )(x0_t.reshape(-1), packed, w)
    xs_t = xs_flat.reshape(K_HOPS + 1, n_class, n_nodes)

    out_t = pl.pallas_call(
        _comb_body,
        out_shape=jax.ShapeDtypeStruct((n_class, n_nodes), jnp.float32),
    )(xs_t, Wa, ba.reshape(1, 1))
    return out_t.T


# speedup vs baseline: 1.2368x; 1.2368x over previous
import functools

import jax
import jax.numpy as jnp
from jax import lax
from jax.experimental import pallas as pl
from jax.experimental.pallas import tpu as pltpu
from jax.experimental.pallas import tpu_sc as plsc

K_HOPS = 10


def _mlp_body(ft_ref, w1_ref, b1_ref, w2_ref, b2_ref, o_ref):
    h = jnp.dot(w1_ref[...], ft_ref[...], preferred_element_type=jnp.float32)
    h = jnp.maximum(h + b1_ref[...], 0.0)
    o_ref[...] = (
        jnp.dot(w2_ref[...], h, preferred_element_type=jnp.float32) + b2_ref[...]
    )


def _comb_body(xs_ref, wa_ref, ba_ref, o_ref):
    xs = xs_ref[...]
    wa = wa_ref[...]
    s = jnp.sum(xs * wa[0][None, :, None], axis=1) + ba_ref[0, 0]
    score = jax.nn.sigmoid(s)
    out = jnp.sum(score[:, None, :] * xs, axis=0)
    m = jnp.max(out, axis=0, keepdims=True)
    lse = jnp.log(jnp.sum(jnp.exp(out - m), axis=0, keepdims=True))
    o_ref[...] = out - m - lse


def _make_sc_body(n_nodes, n_feat_tile, n_chunks, chunk, num_cores, n_tiles):
    slab = n_feat_tile * n_nodes
    n_pairs = n_chunks // 2
    shift = (n_nodes - 1).bit_length()

    def sc_body(x0_hbm, pk_hbm, w_hbm, xs_hbm, xa, xb, pkb, wb, sem0, sem1, sem_out):
        wid = lax.axis_index("s") * num_cores + lax.axis_index("c")
        base = wid * slab
        sems = (sem0, sem1)

        def wait_out(buf, k):
            pltpu.make_async_copy(
                buf, xs_hbm.at[pl.ds(k * (n_tiles * slab) + base, slab)], sem_out
            ).wait()

        def start(c, b):
            off = c * chunk
            boff = b * chunk
            pltpu.async_copy(
                pk_hbm.at[pl.ds(off, chunk)], pkb.at[pl.ds(boff, chunk)], sems[b]
            )
            pltpu.async_copy(
                w_hbm.at[pl.ds(off, chunk)], wb.at[pl.ds(boff, chunk)], sems[b]
            )

        def wait(b):
            boff = b * chunk
            for hbm, buf in ((pk_hbm, pkb), (w_hbm, wb)):
                pltpu.make_async_copy(
                    hbm.at[pl.ds(0, chunk)], buf.at[pl.ds(boff, chunk)], sems[b]
                ).wait()

        pltpu.sync_copy(x0_hbm.at[pl.ds(base, slab)], xa)
        pltpu.async_copy(xa, xs_hbm.at[pl.ds(base, slab)], sem_out)

        start(0, 0)
        start(1, 1)

        for k in range(1, K_HOPS + 1):
            src, dst = (xa, xb) if k % 2 == 1 else (xb, xa)

            srcf = [src.at[pl.ds(f * n_nodes, n_nodes)] for f in range(n_feat_tile)]
            dstf = [dst.at[pl.ds(f * n_nodes, n_nodes)] for f in range(n_feat_tile)]

            @plsc.parallel_loop(0, slab // 16, unroll=8)
            def _zero(i, dst=dst):
                dst[pl.ds(i * 16, 16)] = jnp.zeros((16,), jnp.float32)

            @pl.loop(0, n_pairs)
            def _pair(p, srcf=srcf, dstf=dstf):
                for b in range(2):
                    wait(b)

                    @plsc.parallel_loop(0, chunk // 16, unroll=16)
                    def _grp(g, srcf=srcf, dstf=dstf, b=b):
                        gb = g * 16 + b * chunk
                        pk16 = pkb[pl.ds(gb, 16)]
                        w16 = wb[pl.ds(gb, 16)]
                        col16 = lax.bitwise_and(pk16, (1 << shift) - 1)
                        row16 = lax.shift_right_logical(pk16, shift)
                        for f in range(n_feat_tile):
                            v = plsc.load_gather(srcf[f], [col16])
                            plsc.addupdate_scatter(dstf[f], [row16], v * w16)

                    start(lax.rem(2 * p + 2 + b, n_chunks), b)

            wait_out(src, k - 1)
            pltpu.async_copy(
                dst, xs_hbm.at[pl.ds(k * (n_tiles * slab) + base, slab)], sem_out
            )

        wait_out(xa if K_HOPS % 2 == 0 else xb, K_HOPS)
        wait(0)
        wait(1)

    return sc_body


def kernel(feature, edge_index, edge_weight, W1, b1, W2, b2, Wa, ba):
    n_nodes, _ = feature.shape
    n_class = W2.shape[0]
    n_edges = edge_weight.shape[0]

    f_t = feature.T
    x0_t = pl.pallas_call(
        _mlp_body,
        out_shape=jax.ShapeDtypeStruct((n_class, n_nodes), jnp.float32),
    )(f_t, W1, b1[:, None], W2, b2[:, None])

    num_cores, num_subcores = 2, 16
    n_tiles = num_cores * num_subcores
    n_feat_tile = n_class // n_tiles

    chunk = 16000
    n_chunks = n_edges // chunk
    slab = n_feat_tile * n_nodes

    row = edge_index[0].astype(jnp.int32)
    col = edge_index[1].astype(jnp.int32)
    shift = (n_nodes - 1).bit_length()
    packed = row * (1 << shift) + col
    w = edge_weight.astype(jnp.float32)

    mesh = plsc.VectorSubcoreMesh(core_axis_name="c", subcore_axis_name="s")
    xs_flat = pl.kernel(
        _make_sc_body(n_nodes, n_feat_tile, n_chunks, chunk, num_cores, n_tiles),
        out_type=jax.ShapeDtypeStruct(((K_HOPS + 1) * n_class * n_nodes,), jnp.float32),
        mesh=mesh,
        compiler_params=pltpu.CompilerParams(needs_layout_passes=False),
        scratch_types=[
            pltpu.VMEM((slab,), jnp.float32),
            pltpu.VMEM((slab,), jnp.float32),
            pltpu.VMEM((2 * chunk,), jnp.int32),
            pltpu.VMEM((2 * chunk,), jnp.float32),
            pltpu.SemaphoreType.DMA,
            pltpu.SemaphoreType.DMA,
            pltpu.SemaphoreType.DMA,
        ],
    )(x0_t.reshape(-1), packed, w)
    xs_t = xs_flat.reshape(K_HOPS + 1, n_class, n_nodes)

    out_t = pl.pallas_call(
        _comb_body,
        out_shape=jax.ShapeDtypeStruct((n_class, n_nodes), jnp.float32),
    )(xs_t, Wa, ba.reshape(1, 1))
    return out_t.T


# speedup vs baseline: 1.2756x; 1.0313x over previous
import functools

import jax
import jax.numpy as jnp
from jax import lax
from jax.experimental import pallas as pl
from jax.experimental.pallas import tpu as pltpu
from jax.experimental.pallas import tpu_sc as plsc

K_HOPS = 10


def _mlp_body(ft_ref, w1_ref, b1_ref, w2_ref, b2_ref, o_ref):
    h = jnp.dot(w1_ref[...], ft_ref[...], preferred_element_type=jnp.float32)
    h = jnp.maximum(h + b1_ref[...], 0.0)
    o_ref[...] = (
        jnp.dot(w2_ref[...], h, preferred_element_type=jnp.float32) + b2_ref[...]
    )


def _comb_body(xs_ref, wa_ref, ba_ref, o_ref):
    xs = xs_ref[...]
    wa = wa_ref[...]
    s = jnp.sum(xs * wa[0][None, :, None], axis=1) + ba_ref[0, 0]
    score = jax.nn.sigmoid(s)
    out = jnp.sum(score[:, None, :] * xs, axis=0)
    m = jnp.max(out, axis=0, keepdims=True)
    lse = jnp.log(jnp.sum(jnp.exp(out - m), axis=0, keepdims=True))
    o_ref[...] = out - m - lse


def _make_sc_body(n_nodes, n_feat_tile, n_chunks, chunk, num_cores, n_tiles):
    slab = n_feat_tile * n_nodes
    n_pairs = n_chunks // 2
    shift = (n_nodes - 1).bit_length()

    def sc_body(x0_hbm, pk_hbm, w_hbm, xs_hbm, xa, xb, pkb, wb, sem0, sem1, sem_out):
        wid = lax.axis_index("s") * num_cores + lax.axis_index("c")
        base = wid * slab
        sems = (sem0, sem1)

        def wait_out(buf, k):
            pltpu.make_async_copy(
                buf, xs_hbm.at[pl.ds(k * (n_tiles * slab) + base, slab)], sem_out
            ).wait()

        def start(c, b):
            off = c * chunk
            boff = b * chunk
            pltpu.async_copy(
                pk_hbm.at[pl.ds(off, chunk)], pkb.at[pl.ds(boff, chunk)], sems[b]
            )
            pltpu.async_copy(
                w_hbm.at[pl.ds(off, chunk)], wb.at[pl.ds(boff, chunk)], sems[b]
            )

        def wait(b):
            boff = b * chunk
            for hbm, buf in ((pk_hbm, pkb), (w_hbm, wb)):
                pltpu.make_async_copy(
                    hbm.at[pl.ds(0, chunk)], buf.at[pl.ds(boff, chunk)], sems[b]
                ).wait()

        pltpu.sync_copy(x0_hbm.at[pl.ds(base, slab)], xa)
        pltpu.async_copy(xa, xs_hbm.at[pl.ds(base, slab)], sem_out)

        start(0, 0)
        start(1, 1)

        for k in range(1, K_HOPS + 1):
            src, dst = (xa, xb) if k % 2 == 1 else (xb, xa)

            srcf = [src.at[pl.ds(f * n_nodes, n_nodes)] for f in range(n_feat_tile)]
            dstf = [dst.at[pl.ds(f * n_nodes, n_nodes)] for f in range(n_feat_tile)]

            @plsc.parallel_loop(0, slab // 16, unroll=8)
            def _zero(i, dst=dst):
                dst[pl.ds(i * 16, 16)] = jnp.zeros((16,), jnp.float32)

            @pl.loop(0, n_pairs)
            def _pair(p, srcf=srcf, dstf=dstf):
                for b in range(2):
                    wait(b)

                    @plsc.parallel_loop(0, chunk // 16, unroll=20)
                    def _grp(g, srcf=srcf, dstf=dstf, b=b):
                        gb = g * 16 + b * chunk
                        pk16 = pkb[pl.ds(gb, 16)]
                        w16 = wb[pl.ds(gb, 16)]
                        col16 = lax.bitwise_and(pk16, (1 << shift) - 1)
                        row16 = lax.shift_right_logical(pk16, shift)
                        for f in range(n_feat_tile):
                            v = plsc.load_gather(srcf[f], [col16])
                            plsc.addupdate_scatter(dstf[f], [row16], v * w16)

                    start(lax.rem(2 * p + 2 + b, n_chunks), b)

            wait_out(src, k - 1)
            pltpu.async_copy(
                dst, xs_hbm.at[pl.ds(k * (n_tiles * slab) + base, slab)], sem_out
            )

        wait_out(xa if K_HOPS % 2 == 0 else xb, K_HOPS)
        wait(0)
        wait(1)

    return sc_body


def kernel(feature, edge_index, edge_weight, W1, b1, W2, b2, Wa, ba):
    n_nodes, _ = feature.shape
    n_class = W2.shape[0]
    n_edges = edge_weight.shape[0]

    f_t = feature.T
    x0_t = pl.pallas_call(
        _mlp_body,
        out_shape=jax.ShapeDtypeStruct((n_class, n_nodes), jnp.float32),
    )(f_t, W1, b1[:, None], W2, b2[:, None])

    num_cores, num_subcores = 2, 16
    n_tiles = num_cores * num_subcores
    n_feat_tile = n_class // n_tiles

    chunk = 16000
    n_chunks = n_edges // chunk
    slab = n_feat_tile * n_nodes

    row = edge_index[0].astype(jnp.int32)
    col = edge_index[1].astype(jnp.int32)
    shift = (n_nodes - 1).bit_length()
    packed = row * (1 << shift) + col
    w = edge_weight.astype(jnp.float32)

    mesh = plsc.VectorSubcoreMesh(core_axis_name="c", subcore_axis_name="s")
    xs_flat = pl.kernel(
        _make_sc_body(n_nodes, n_feat_tile, n_chunks, chunk, num_cores, n_tiles),
        out_type=jax.ShapeDtypeStruct(((K_HOPS + 1) * n_class * n_nodes,), jnp.float32),
        mesh=mesh,
        compiler_params=pltpu.CompilerParams(needs_layout_passes=False),
        scratch_types=[
            pltpu.VMEM((slab,), jnp.float32),
            pltpu.VMEM((slab,), jnp.float32),
            pltpu.VMEM((2 * chunk,), jnp.int32),
            pltpu.VMEM((2 * chunk,), jnp.float32),
            pltpu.SemaphoreType.DMA,
            pltpu.SemaphoreType.DMA,
            pltpu.SemaphoreType.DMA,
        ],
    )(x0_t.reshape(-1), packed, w)
    xs_t = xs_flat.reshape(K_HOPS + 1, n_class, n_nodes)

    out_t = pl.pallas_call(
        _comb_body,
        out_shape=jax.ShapeDtypeStruct((n_class, n_nodes), jnp.float32),
    )(xs_t, Wa, ba.reshape(1, 1))
    return out_t.T
